# baseline (speedup 1.0000x reference)
import functools

import jax
import jax.numpy as jnp
from jax import lax
from jax.experimental import pallas as pl
from jax.experimental.pallas import tpu as pltpu

N_LAB = 10
T = 20


def _body(lab_sref, x_ref, rates_ref, assign_ref, acc_ref, s_ref, cnt_ref,
          *, grid, bb):
    i = pl.program_id(0)

    @pl.when(i == 0)
    def _init():
        acc_ref[...] = jnp.zeros_like(acc_ref)
        for l in range(N_LAB):
            cnt_ref[l] = 0

    groups = []
    for g in range(T // 4):
        gs = x_ref[4 * g]
        for t in range(4 * g + 1, 4 * g + 4):
            gs = gs + x_ref[t]
        groups.append(gs)
    s = groups[0]
    for g in range(1, T // 4):
        s = s + groups[g]

    s_ref[...] = s.reshape(bb, 8, 128)

    zero = jnp.zeros((8, 128), jnp.float32)

    def body(b, accs):
        lab = lab_sref[i * bb + b]
        cnt_ref[lab] = cnt_ref[lab] + 1
        row = s_ref[b]
        return tuple(
            accs[l] + jnp.where(lab == l, row, zero) for l in range(N_LAB))

    accs0 = tuple(acc_ref[l] for l in range(N_LAB))
    accs = lax.fori_loop(0, bb, body, accs0)
    for l in range(N_LAB):
        acc_ref[l] = accs[l]

    @pl.when(i == grid - 1)
    def _finish():
        means = []
        for l in range(N_LAB):
            c_l = cnt_ref[l]
            m_l = acc_ref[l] / jnp.maximum(c_l.astype(jnp.float32), 1.0)
            m_l = jnp.where(c_l > 0, m_l, 0.0)
            means.append(m_l)
            rates_ref[l] = m_l
        m = means[0]
        am = jnp.zeros(m.shape, dtype=jnp.int32)
        for l in range(1, N_LAB):
            gt = means[l] > m
            am = jnp.where(gt, l, am)
            m = jnp.where(gt, means[l], m)
        assign_ref[...] = am


@jax.jit
def kernel(spikes, labels):
    b, t, n = spikes.shape
    xt = jnp.transpose(spikes, (1, 0, 2))

    grid = 64
    bb = b // grid

    grid_spec = pltpu.PrefetchScalarGridSpec(
        num_scalar_prefetch=1,
        grid=(grid,),
        in_specs=[
            pl.BlockSpec((t, bb, n), lambda i, *_: (0, i, 0)),
        ],
        out_specs=[
            pl.BlockSpec((N_LAB, 8, 128), lambda i, *_: (0, 0, 0)),
            pl.BlockSpec((8, 128), lambda i, *_: (0, 0)),
        ],
        scratch_shapes=[
            pltpu.VMEM((N_LAB, 8, 128), jnp.float32),
            pltpu.VMEM((bb, 8, 128), jnp.float32),
            pltpu.SMEM((N_LAB,), jnp.int32),
        ],
    )

    rates3, assign2 = pl.pallas_call(
        functools.partial(_body, grid=grid, bb=bb),
        grid_spec=grid_spec,
        out_shape=[
            jax.ShapeDtypeStruct((N_LAB, 8, 128), jnp.float32),
            jax.ShapeDtypeStruct((8, 128), jnp.int32),
        ],
    )(labels, xt)

    rates = rates3.reshape(N_LAB, n).T
    assignments = assign2.reshape(n)
    return assignments, rates


# speedup vs baseline: 1.0524x; 1.0524x over previous
import functools

import jax
import jax.numpy as jnp
from jax import lax
from jax.experimental import pallas as pl
from jax.experimental.pallas import tpu as pltpu

N_LAB = 10
T = 20


def _body(lab_sref, x_ref, rates_ref, assign_ref, acc_ref, s_ref, cnt_ref,
          *, grid, bb):
    i = pl.program_id(0)

    @pl.when(i == 0)
    def _init():
        acc_ref[...] = jnp.zeros_like(acc_ref)
        for l in range(N_LAB):
            cnt_ref[l] = 0

    groups = []
    for g in range(T // 4):
        gs = x_ref[4 * g]
        for t in range(4 * g + 1, 4 * g + 4):
            gs = gs + x_ref[t]
        groups.append(gs)
    s = groups[0]
    for g in range(1, T // 4):
        s = s + groups[g]

    s_ref[...] = s.reshape(bb, 8, 128)

    zero = jnp.zeros((8, 128), jnp.float32)

    def body(b, accs):
        lab = lab_sref[i * bb + b]
        cnt_ref[lab] = cnt_ref[lab] + 1
        row = s_ref[b]
        return tuple(
            accs[l] + jnp.where(lab == l, row, zero) for l in range(N_LAB))

    accs0 = tuple(acc_ref[l] for l in range(N_LAB))
    accs = lax.fori_loop(0, bb, body, accs0)
    for l in range(N_LAB):
        acc_ref[l] = accs[l]

    @pl.when(i == grid - 1)
    def _finish():
        means = []
        for l in range(N_LAB):
            c_l = cnt_ref[l]
            m_l = acc_ref[l] / jnp.maximum(c_l.astype(jnp.float32), 1.0)
            m_l = jnp.where(c_l > 0, m_l, 0.0)
            means.append(m_l)
            rates_ref[l] = m_l
        m = means[0]
        am = jnp.zeros(m.shape, dtype=jnp.int32)
        for l in range(1, N_LAB):
            gt = means[l] > m
            am = jnp.where(gt, l, am)
            m = jnp.where(gt, means[l], m)
        assign_ref[...] = am


@jax.jit
def kernel(spikes, labels):
    b, t, n = spikes.shape
    xt = jnp.transpose(spikes, (1, 0, 2))

    grid = 16
    bb = b // grid

    grid_spec = pltpu.PrefetchScalarGridSpec(
        num_scalar_prefetch=1,
        grid=(grid,),
        in_specs=[
            pl.BlockSpec((t, bb, n), lambda i, *_: (0, i, 0)),
        ],
        out_specs=[
            pl.BlockSpec((N_LAB, 8, 128), lambda i, *_: (0, 0, 0)),
            pl.BlockSpec((8, 128), lambda i, *_: (0, 0)),
        ],
        scratch_shapes=[
            pltpu.VMEM((N_LAB, 8, 128), jnp.float32),
            pltpu.VMEM((bb, 8, 128), jnp.float32),
            pltpu.SMEM((N_LAB,), jnp.int32),
        ],
    )

    rates3, assign2 = pl.pallas_call(
        functools.partial(_body, grid=grid, bb=bb),
        grid_spec=grid_spec,
        out_shape=[
            jax.ShapeDtypeStruct((N_LAB, 8, 128), jnp.float32),
            jax.ShapeDtypeStruct((8, 128), jnp.int32),
        ],
    )(labels, xt)

    rates = rates3.reshape(N_LAB, n).T
    assignments = assign2.reshape(n)
    return assignments, rates


# speedup vs baseline: 1.0595x; 1.0067x over previous
import functools

import jax
import jax.numpy as jnp
from jax import lax
from jax.experimental import pallas as pl
from jax.experimental.pallas import tpu as pltpu

N_LAB = 10
T = 20


def _body(lab_sref, x_ref, rates_ref, assign_ref, acc_ref, s_ref, cnt_ref,
          *, grid, bb):
    i = pl.program_id(0)

    @pl.when(i == 0)
    def _init():
        acc_ref[...] = jnp.zeros_like(acc_ref)
        for l in range(N_LAB):
            cnt_ref[l] = 0

    groups = []
    for g in range(T // 4):
        gs = x_ref[4 * g]
        for t in range(4 * g + 1, 4 * g + 4):
            gs = gs + x_ref[t]
        groups.append(gs)
    s = groups[0]
    for g in range(1, T // 4):
        s = s + groups[g]

    s_ref[...] = s.reshape(bb, 8, 128)

    zero = jnp.zeros((8, 128), jnp.float32)

    def body(b, accs):
        lab = lab_sref[i * bb + b]
        cnt_ref[lab] = cnt_ref[lab] + 1
        row = s_ref[b]
        return tuple(
            accs[l] + jnp.where(lab == l, row, zero) for l in range(N_LAB))

    accs0 = tuple(acc_ref[l] for l in range(N_LAB))
    accs = lax.fori_loop(0, bb, body, accs0)
    for l in range(N_LAB):
        acc_ref[l] = accs[l]

    @pl.when(i == grid - 1)
    def _finish():
        means = []
        for l in range(N_LAB):
            c_l = cnt_ref[l]
            m_l = acc_ref[l] / jnp.maximum(c_l.astype(jnp.float32), 1.0)
            m_l = jnp.where(c_l > 0, m_l, 0.0)
            means.append(m_l)
            rates_ref[l] = m_l
        m = means[0]
        am = jnp.zeros(m.shape, dtype=jnp.int32)
        for l in range(1, N_LAB):
            gt = means[l] > m
            am = jnp.where(gt, l, am)
            m = jnp.where(gt, means[l], m)
        assign_ref[...] = am


@jax.jit
def kernel(spikes, labels):
    b, t, n = spikes.shape
    xt = jnp.transpose(spikes, (1, 0, 2))

    grid = 32
    bb = b // grid

    grid_spec = pltpu.PrefetchScalarGridSpec(
        num_scalar_prefetch=1,
        grid=(grid,),
        in_specs=[
            pl.BlockSpec((t, bb, n), lambda i, *_: (0, i, 0)),
        ],
        out_specs=[
            pl.BlockSpec((N_LAB, 8, 128), lambda i, *_: (0, 0, 0)),
            pl.BlockSpec((8, 128), lambda i, *_: (0, 0)),
        ],
        scratch_shapes=[
            pltpu.VMEM((N_LAB, 8, 128), jnp.float32),
            pltpu.VMEM((bb, 8, 128), jnp.float32),
            pltpu.SMEM((N_LAB,), jnp.int32),
        ],
    )

    rates3, assign2 = pl.pallas_call(
        functools.partial(_body, grid=grid, bb=bb),
        grid_spec=grid_spec,
        out_shape=[
            jax.ShapeDtypeStruct((N_LAB, 8, 128), jnp.float32),
            jax.ShapeDtypeStruct((8, 128), jnp.int32),
        ],
    )(labels, xt)

    rates = rates3.reshape(N_LAB, n).T
    assignments = assign2.reshape(n)
    return assignments, rates
